# baseline (speedup 1.0000x reference)
import functools

import jax
import jax.numpy as jnp
from jax.experimental import pallas as pl
from jax.experimental.pallas import tpu as pltpu

_N = 10000
_NH = 128

_IB = 80
_SPS = 5
_RPS = _IB * _SPS
_NI = _N // _RPS


def _agg_body(*refs, n_i):
    adj_refs = refs[:_SPS]
    x_ref, w_ref, b_ref, o_ref, acc_ref, deg_ref = refs[_SPS:]
    i = pl.program_id(0)

    @pl.when(i == 0)
    def _zero():
        acc_ref[...] = jnp.zeros_like(acc_ref)
        deg_ref[...] = jnp.zeros_like(deg_ref)

    sup = jnp.maximum(
        jax.lax.dot_general(
            x_ref[...], w_ref[...], (((1,), (1,)), ((), ())),
            preferred_element_type=jnp.float32) + b_ref[...],
        0.0).astype(jnp.bfloat16)

    deg = deg_ref[...]
    acc = acc_ref[...]
    for s, stripe_ref in enumerate(adj_refs):
        sel = jnp.where(stripe_ref[...] > 0.0, 1.0, 0.0)
        deg += jnp.sum(sel, axis=0, keepdims=True)
        mask = sel.astype(jnp.bfloat16)
        acc += jax.lax.dot_general(
            sup[s * _IB:(s + 1) * _IB, :], mask, (((0,), (0,)), ((), ())),
            preferred_element_type=jnp.float32)
    deg_ref[...] = deg
    acc_ref[...] = acc

    @pl.when(i == n_i - 1)
    def _emit():
        o_ref[...] = jnp.transpose(acc_ref[...] / deg_ref[...])


def kernel(input, adj, W, b):
    adj_specs = [
        pl.BlockSpec((_IB, _N), functools.partial(
            lambda s, i: (_SPS * i + s, 0), s))
        for s in range(_SPS)
    ]
    return pl.pallas_call(
        functools.partial(_agg_body, n_i=_NI),
        grid=(_NI,),
        in_specs=adj_specs + [
            pl.BlockSpec((_RPS, _NH), lambda i: (i, 0)),
            pl.BlockSpec((_NH, _NH), lambda i: (0, 0)),
            pl.BlockSpec((1, _NH), lambda i: (0, 0)),
        ],
        out_specs=pl.BlockSpec((_N, _NH), lambda i: (0, 0)),
        out_shape=jax.ShapeDtypeStruct((_N, _NH), jnp.float32),
        scratch_shapes=[
            pltpu.VMEM((_NH, _N), jnp.float32),
            pltpu.VMEM((1, _N), jnp.float32),
        ],
        compiler_params=pltpu.CompilerParams(
            dimension_semantics=("arbitrary",),
        ),
    )(*([adj] * _SPS), input, W, b.reshape(1, _NH))


# speedup vs baseline: 1.0681x; 1.0681x over previous
import jax
import jax.numpy as jnp
from jax.experimental import pallas as pl
from jax.experimental.pallas import tpu as pltpu

_N = 10000
_NH = 128

_IB = 200
_NI = _N // _IB
_NBUF = 4


def _agg_body(adj_ref, x_ref, w_ref, b_ref, o_ref,
              buf_ref, acc_ref, deg_ref, sem):
    def start_copy(k, slot):
        pltpu.make_async_copy(
            adj_ref.at[pl.ds(k * _IB, _IB), :],
            buf_ref.at[slot],
            sem.at[slot],
        ).start()

    for k in range(_NBUF):
        start_copy(k, k)

    acc_ref[...] = jnp.zeros_like(acc_ref)
    deg_ref[...] = jnp.zeros_like(deg_ref)

    for k in range(_NI):
        slot = k % _NBUF
        pltpu.make_async_copy(
            adj_ref.at[pl.ds(k * _IB, _IB), :],
            buf_ref.at[slot],
            sem.at[slot],
        ).wait()
        sup = jnp.maximum(
            jax.lax.dot_general(
                x_ref[k * _IB:(k + 1) * _IB, :], w_ref[...],
                (((1,), (1,)), ((), ())),
                preferred_element_type=jnp.float32) + b_ref[...],
            0.0).astype(jnp.bfloat16)
        sel = jnp.where(buf_ref[slot] > 0.0, 1.0, 0.0)
        deg_ref[...] += jnp.sum(sel, axis=0, keepdims=True)
        mask = sel.astype(jnp.bfloat16)
        acc_ref[...] += jax.lax.dot_general(
            sup, mask, (((0,), (0,)), ((), ())),
            preferred_element_type=jnp.float32)
        if k + _NBUF < _NI:
            start_copy(k + _NBUF, slot)

    o_ref[...] = jnp.transpose(acc_ref[...] / deg_ref[...])


def kernel(input, adj, W, b):
    return pl.pallas_call(
        _agg_body,
        in_specs=[
            pl.BlockSpec(memory_space=pl.ANY),
            pl.BlockSpec(memory_space=pltpu.MemorySpace.VMEM),
            pl.BlockSpec(memory_space=pltpu.MemorySpace.VMEM),
            pl.BlockSpec(memory_space=pltpu.MemorySpace.VMEM),
        ],
        out_specs=pl.BlockSpec(memory_space=pltpu.MemorySpace.VMEM),
        out_shape=jax.ShapeDtypeStruct((_N, _NH), jnp.float32),
        scratch_shapes=[
            pltpu.VMEM((_NBUF, _IB, _N), jnp.float32),
            pltpu.VMEM((_NH, _N), jnp.float32),
            pltpu.VMEM((1, _N), jnp.float32),
            pltpu.SemaphoreType.DMA((_NBUF,)),
        ],
    )(adj, input, W, b.reshape(1, _NH))
